# baseline (speedup 1.0000x reference)
import functools

import jax
import jax.numpy as jnp
from jax import lax
from jax.experimental import pallas as pl
from jax.experimental.pallas import tpu as pltpu
from jax.experimental.pallas import tpu_sc as plsc

C, H, W, S = 96, 512, 512, 128
R_BLK = 6144

_sc_mesh = plsc.VectorSubcoreMesh(core_axis_name="c", subcore_axis_name="s")


@functools.partial(
    pl.kernel,
    mesh=_sc_mesh,
    out_type=[
        jax.ShapeDtypeStruct((W,), jnp.float32),
        jax.ShapeDtypeStruct((W,), jnp.float32),
    ],
    scratch_types=[
        pltpu.VMEM((S,), jnp.int32),
        pltpu.VMEM((W,), jnp.float32),
        pltpu.VMEM((S,), jnp.float32),
    ],
)
def _build_ab(sites_hbm, a_hbm, b_hbm, sites_v, init_v, vals_v):
    wid = lax.axis_index("s") * 2 + lax.axis_index("c")

    @pl.when(wid == 0)
    def _():
        pltpu.sync_copy(sites_hbm, sites_v)
        ones = jnp.full((16,), 1.0, jnp.float32)
        zeros = jnp.zeros((16,), jnp.float32)
        neg = jnp.full((16,), -1.0, jnp.float32)
        for i in range(W // 16):
            init_v[pl.ds(i * 16, 16)] = ones
        pltpu.sync_copy(init_v, a_hbm)
        for i in range(W // 16):
            init_v[pl.ds(i * 16, 16)] = zeros
        pltpu.sync_copy(init_v, b_hbm)
        for j in range(S // 16):
            vals_v[pl.ds(j * 16, 16)] = neg
        pltpu.sync_copy(vals_v, a_hbm.at[sites_v])
        for j in range(S // 16):
            vals_v[pl.ds(j * 16, 16)] = ones
        pltpu.sync_copy(vals_v, b_hbm.at[sites_v])


def _flip_body(a_ref, b_ref, x_ref, o_ref):
    rep = R_BLK // 8
    o_ref[...] = x_ref[...] * jnp.tile(a_ref[...], (rep, 1)) + jnp.tile(
        b_ref[...], (rep, 1))


def kernel(x, mask_sites):
    a1, b1 = _build_ab(mask_sites)
    a = jnp.broadcast_to(a1.reshape(1, W), (8, W))
    b = jnp.broadcast_to(b1.reshape(1, W), (8, W))
    x2 = x.reshape(C * H, W)
    out = pl.pallas_call(
        _flip_body,
        grid=((C * H) // R_BLK,),
        in_specs=[
            pl.BlockSpec((8, W), lambda i: (0, 0)),
            pl.BlockSpec((8, W), lambda i: (0, 0)),
            pl.BlockSpec((R_BLK, W), lambda i: (i, 0)),
        ],
        out_specs=pl.BlockSpec((R_BLK, W), lambda i: (i, 0)),
        out_shape=jax.ShapeDtypeStruct((C * H, W), jnp.float32),
    )(a, b, x2)
    return out.reshape(C, H, W)


# speedup vs baseline: 1.0324x; 1.0324x over previous
import functools

import jax
import jax.numpy as jnp
from jax import lax
from jax.experimental import pallas as pl
from jax.experimental.pallas import tpu as pltpu
from jax.experimental.pallas import tpu_sc as plsc

C, H, W, S = 96, 512, 512, 128
R_BLK = 6144

_sc_mesh = plsc.VectorSubcoreMesh(core_axis_name="c", subcore_axis_name="s")


@functools.partial(
    pl.kernel,
    mesh=_sc_mesh,
    out_type=jax.ShapeDtypeStruct((W,), jnp.float32),
    scratch_types=[
        pltpu.VMEM((S,), jnp.int32),
        pltpu.VMEM((W,), jnp.float32),
        pltpu.VMEM((S,), jnp.float32),
        pltpu.SemaphoreType.DMA,
        pltpu.SemaphoreType.DMA,
    ],
)
def _build_mask(sites_hbm, m_hbm, sites_v, zeros_v, ones_v, sem1, sem2):
    wid = lax.axis_index("s") * 2 + lax.axis_index("c")

    @pl.when(wid == 0)
    def _():
        ones = jnp.full((16,), 1.0, jnp.float32)
        zeros = jnp.zeros((16,), jnp.float32)
        for i in range(W // 16):
            zeros_v[pl.ds(i * 16, 16)] = zeros
        for j in range(S // 16):
            ones_v[pl.ds(j * 16, 16)] = ones
        cp_sites = pltpu.async_copy(sites_hbm, sites_v, sem1)
        cp_init = pltpu.async_copy(zeros_v, m_hbm, sem2)
        cp_sites.wait()
        cp_init.wait()
        pltpu.sync_copy(ones_v, m_hbm.at[sites_v])


def _flip_body(m_ref, x_ref, o_ref, a_ref, b_ref):
    @pl.when(pl.program_id(0) == 0)
    def _build_tables():
        m = m_ref[...]
        a_ref[...] = jnp.broadcast_to(1.0 - 2.0 * m, (8, W))
        b_ref[...] = jnp.broadcast_to(m, (8, W))

    rep = R_BLK // 8
    o_ref[...] = x_ref[...] * jnp.tile(a_ref[...], (rep, 1)) + jnp.tile(
        b_ref[...], (rep, 1))


def kernel(x, mask_sites):
    m = _build_mask(mask_sites)
    x2 = x.reshape(C * H, W)
    out = pl.pallas_call(
        _flip_body,
        grid=((C * H) // R_BLK,),
        in_specs=[
            pl.BlockSpec((1, W), lambda i: (0, 0)),
            pl.BlockSpec((R_BLK, W), lambda i: (i, 0)),
        ],
        out_specs=pl.BlockSpec((R_BLK, W), lambda i: (i, 0)),
        out_shape=jax.ShapeDtypeStruct((C * H, W), jnp.float32),
        scratch_shapes=[
            pltpu.VMEM((8, W), jnp.float32),
            pltpu.VMEM((8, W), jnp.float32),
        ],
    )(m.reshape(1, W), x2)
    return out.reshape(C, H, W)
